# baseline (speedup 1.0000x reference)
import jax
import jax.numpy as jnp
from jax.experimental import pallas as pl
from jax.experimental.pallas import tpu as pltpu

P = 8
ENC = 64
H = 512
B = 1024
E = P * ENC + P
TILE = 256
NT = 5
BP = B + TILE


def _encode_rows(obs, freq):
    parts = []
    for p_ in range(P):
        f = obs[:, p_:p_ + 1] * freq
        parts.append(jnp.cos(f))
        parts.append(jnp.sin(f))
    parts.append(obs[:, P:2 * P])
    return jnp.concatenate(parts, axis=1)


def _route_kernel(obs_ref, freq_ref, enc_ref, starts_ref, pos_ref):
    obs = obs_ref[...]
    one_hot = obs[:, P:2 * P]
    col8 = jax.lax.broadcasted_iota(jnp.int32, (B, P), 1)
    mx = jnp.max(one_hot, axis=1, keepdims=True)
    idx = jnp.where(one_hot == mx, col8, P)
    pid = jnp.min(idx, axis=1, keepdims=True)
    mt = (pid == col8).astype(jnp.float32)

    ri = jax.lax.broadcasted_iota(jnp.int32, (B, B), 0)
    ci = jax.lax.broadcasted_iota(jnp.int32, (B, B), 1)
    tri = (ci <= ri).astype(jnp.float32)
    csum = jnp.dot(tri, mt, preferred_element_type=jnp.float32)
    counts = csum[B - 1:B, :]

    qi = jax.lax.broadcasted_iota(jnp.int32, (P, 16), 0)
    pi_ = jax.lax.broadcasted_iota(jnp.int32, (P, 16), 1)
    stm = (qi < pi_).astype(jnp.float32)
    starts16 = jnp.dot(counts, stm, preferred_element_type=jnp.float32)
    starts_ref[...] = starts16.astype(jnp.int32)

    pos = jnp.sum(mt * (csum - 1.0 + starts16[:, :P]), axis=1, keepdims=True)
    pos_ref[...] = pos

    ck = jax.lax.broadcasted_iota(jnp.int32, (B, BP), 1)
    pit = (pos.astype(jnp.int32) == ck).astype(jnp.float32)
    obs_sorted = jax.lax.dot_general(
        pit, obs, (((0,), (0,)), ((), ())),
        preferred_element_type=jnp.float32)
    enc_ref[...] = _encode_rows(obs_sorted, freq_ref[...])


def _expert_kernel(starts_ref, enc_ref,
                   w1, b1, w2, b2, w3, b3, wd, bd,
                   u1, c1, u2, c2, u3, c3, ud, cd,
                   out_ref):
    p = pl.program_id(0)
    t = pl.program_id(1)

    @pl.when((p == 0) & (t == 0))
    def _init():
        out_ref[...] = jnp.zeros_like(out_ref)

    s = starts_ref[p]
    e = starts_ref[p + 1]
    sa = (s // 8) * 8
    lo = sa + t * TILE

    @pl.when(lo < e)
    def _work():
        x = enc_ref[pl.ds(lo, TILE), :]

        def mlp(w1r, b1r, w2r, b2r, w3r, b3r, wdr, bdr):
            h = jnp.maximum(
                jnp.dot(x, w1r[0], preferred_element_type=jnp.float32) + b1r[0], 0.0)
            h = jnp.maximum(
                jnp.dot(h, w2r[0], preferred_element_type=jnp.float32) + b2r[0], 0.0)
            h = jnp.maximum(
                jnp.dot(h, w3r[0], preferred_element_type=jnp.float32) + b3r[0], 0.0)
            o = jnp.dot(h, wdr[0], preferred_element_type=jnp.float32) + bdr[0]
            return jnp.pi * jnp.tanh(o)

        ypi = mlp(w1, b1, w2, b2, w3, b3, wd, bd)
        yvf = mlp(u1, c1, u2, c2, u3, c3, ud, cd)
        y = jnp.concatenate([ypi, yvf], axis=1)

        rows = lo + jax.lax.broadcasted_iota(jnp.int32, (TILE, 2), 0)
        valid = (rows >= s) & (rows < e)
        old = out_ref[pl.ds(lo, TILE), :]
        out_ref[pl.ds(lo, TILE), :] = jnp.where(valid, y, old)


def _unsort_kernel(pos_ref, ys_ref, pi_ref, vf_ref):
    pos = pos_ref[...].astype(jnp.int32)
    ck = jax.lax.broadcasted_iota(jnp.int32, (B, BP), 1)
    oh = (pos == ck).astype(jnp.float32)
    y = jnp.dot(oh, ys_ref[...], preferred_element_type=jnp.float32)
    pi_ref[...] = y[:, 0:1]
    vf_ref[...] = y[:, 1:2]


def kernel(observations, frequencies, pW1, pb1, pW2, pb2, pW3, pb3, pWd, pbd,
           vW1, vb1, vW2, vb2, vW3, vb3, vWd, vbd):
    freq = frequencies.reshape(1, ENC // 2)

    enc_sorted, starts, pos = pl.pallas_call(
        _route_kernel,
        out_shape=[
            jax.ShapeDtypeStruct((BP, E), jnp.float32),
            jax.ShapeDtypeStruct((1, 16), jnp.int32),
            jax.ShapeDtypeStruct((B, 1), jnp.float32),
        ],
    )(observations, freq)

    def wsp(a):
        return pl.BlockSpec((1,) + a.shape[1:], lambda p, t: (p,) + (0,) * (a.ndim - 1))

    weight_args = (pW1, pb1[:, None, :], pW2, pb2[:, None, :],
                   pW3, pb3[:, None, :], pWd, pbd[:, None, :],
                   vW1, vb1[:, None, :], vW2, vb2[:, None, :],
                   vW3, vb3[:, None, :], vWd, vbd[:, None, :])

    ys = pl.pallas_call(
        _expert_kernel,
        grid=(P, NT),
        in_specs=[
            pl.BlockSpec(memory_space=pltpu.SMEM),
            pl.BlockSpec((BP, E), lambda p, t: (0, 0)),
        ] + [wsp(a) for a in weight_args],
        out_specs=pl.BlockSpec((BP, 2), lambda p, t: (0, 0)),
        out_shape=jax.ShapeDtypeStruct((BP, 2), jnp.float32),
    )(starts.reshape(16), enc_sorted, *weight_args)

    latent_pi, latent_vf = pl.pallas_call(
        _unsort_kernel,
        out_shape=[
            jax.ShapeDtypeStruct((B, 1), jnp.float32),
            jax.ShapeDtypeStruct((B, 1), jnp.float32),
        ],
    )(pos, ys)

    return (latent_pi, latent_vf)
